# Initial kernel scaffold; baseline (speedup 1.0000x reference)
import jax
import jax.numpy as jnp
from jax.experimental import pallas as pl


def kernel(x, edge_index, W1, b1, W2, b2, Wd, bd, Wp, bp):
    raise NotImplementedError("write your pallas kernel here")


# speedup vs baseline: 20.1141x; 20.1141x over previous
import functools

import jax
import jax.numpy as jnp
from jax import lax
from jax.experimental import pallas as pl
from jax.experimental.pallas import tpu as pltpu
from jax.experimental.pallas import tpu_sc as plsc

N = 10000
E = 320000
D = 128
NC = 2
NS = 16
NW = NC * NS
CH = 128
C = 80
EPAD = NW * C * CH
NPAD = 10240
RPW = NPAD // NS
ZR = 64


def _mesh():
    return plsc.VectorSubcoreMesh(core_axis_name="c", subcore_axis_name="s")


def _sc_degree(dst_rows):
    @functools.partial(
        pl.kernel,
        out_type=jax.ShapeDtypeStruct((NC * NPAD, D), jnp.float32),
        mesh=_mesh(),
        scratch_types=[
            pltpu.VMEM((C, CH), jnp.int32),
            pltpu.VMEM((CH, D), jnp.float32),
            pltpu.VMEM((ZR, D), jnp.float32),
            pltpu.VMEM_SHARED((NPAD, D), jnp.float32),
        ],
    )
    def deg_kernel(dst_hbm, out_hbm, dstv, onesv, zerov, degs):
        c = lax.axis_index("c")
        s = lax.axis_index("s")
        w = c * NS + s

        @pl.loop(0, CH)
        def _(i):
            @pl.loop(0, D, step=16)
            def _(k):
                onesv[i, pl.ds(k, 16)] = jnp.ones((16,), jnp.float32)

        @pl.loop(0, ZR)
        def _(i):
            @pl.loop(0, D, step=16)
            def _(k):
                zerov[i, pl.ds(k, 16)] = jnp.zeros((16,), jnp.float32)

        @pl.loop(0, RPW // ZR)
        def _(i):
            pltpu.sync_copy(zerov, degs.at[pl.ds(s * RPW + i * ZR, ZR)])

        plsc.subcore_barrier()
        pltpu.sync_copy(dst_hbm.at[pl.ds(w * C, C)], dstv)

        @pl.loop(0, C)
        def _(j):
            pltpu.sync_copy(onesv, degs.at[dstv.at[j]], add=True)

        plsc.subcore_barrier()
        pltpu.sync_copy(
            degs.at[pl.ds(s * RPW, RPW)],
            out_hbm.at[pl.ds(c * NPAD + s * RPW, RPW)],
        )

    return deg_kernel(dst_rows)


def _sc_aggregate(y, src_rows, dst_rows):
    @functools.partial(
        pl.kernel,
        out_type=jax.ShapeDtypeStruct((NC * NPAD, D), jnp.float32),
        mesh=_mesh(),
        scratch_types=[
            pltpu.VMEM((C, CH), jnp.int32),
            pltpu.VMEM((C, CH), jnp.int32),
            pltpu.VMEM((CH, D), jnp.float32),
            pltpu.VMEM((ZR, D), jnp.float32),
            pltpu.VMEM_SHARED((NPAD, D), jnp.float32),
        ],
    )
    def agg_kernel(y_hbm, src_hbm, dst_hbm, out_hbm, srcv, dstv, rowsv, zerov, aggs):
        c = lax.axis_index("c")
        s = lax.axis_index("s")
        w = c * NS + s

        @pl.loop(0, ZR)
        def _(i):
            @pl.loop(0, D, step=16)
            def _(k):
                zerov[i, pl.ds(k, 16)] = jnp.zeros((16,), jnp.float32)

        @pl.loop(0, RPW // ZR)
        def _(i):
            pltpu.sync_copy(zerov, aggs.at[pl.ds(s * RPW + i * ZR, ZR)])

        plsc.subcore_barrier()
        pltpu.sync_copy(src_hbm.at[pl.ds(w * C, C)], srcv)
        pltpu.sync_copy(dst_hbm.at[pl.ds(w * C, C)], dstv)

        @pl.loop(0, C)
        def _(j):
            pltpu.sync_copy(y_hbm.at[srcv.at[j]], rowsv)
            pltpu.sync_copy(rowsv, aggs.at[dstv.at[j]], add=True)

        plsc.subcore_barrier()
        pltpu.sync_copy(
            aggs.at[pl.ds(s * RPW, RPW)],
            out_hbm.at[pl.ds(c * NPAD + s * RPW, RPW)],
        )

    return agg_kernel(y, src_rows, dst_rows)


def _tc_matmul(x, W):
    def body(x_ref, w_ref, o_ref):
        o_ref[...] = jnp.dot(
            x_ref[...], w_ref[...], preferred_element_type=jnp.float32
        )

    return pl.pallas_call(
        body, out_shape=jax.ShapeDtypeStruct((N, D), jnp.float32)
    )(x, W)


def _tc_norm_scale(deg_parts, xw):
    def body(p_ref, xw_ref, dis_ref, y_ref):
        p = p_ref[...]
        deg = 1.0 + p[:N, :1] + p[NPAD:NPAD + N, :1]
        dis_b = jnp.broadcast_to(lax.rsqrt(deg), (N, D))
        dis_ref[...] = dis_b
        y_ref[...] = dis_b * xw_ref[...]

    return pl.pallas_call(
        body,
        out_shape=[
            jax.ShapeDtypeStruct((N, D), jnp.float32),
            jax.ShapeDtypeStruct((N, D), jnp.float32),
        ],
    )(deg_parts, xw)


def _tc_layer(agg_parts, y, dis_b, b, W):
    def body(a_ref, y_ref, d_ref, b_ref, w_ref, o_ref):
        d = d_ref[...]
        a = a_ref[:N, :] + a_ref[NPAD:NPAD + N, :]
        h = jnp.maximum(d * (a + y_ref[...]) + b_ref[...], 0.0)
        o_ref[...] = d * jnp.dot(h, w_ref[...], preferred_element_type=jnp.float32)

    return pl.pallas_call(
        body, out_shape=jax.ShapeDtypeStruct((N, D), jnp.float32)
    )(agg_parts, y, dis_b, b, W)


def _tc_final(agg_parts, y, dis_b, b, Whp, bhp):
    def body(a_ref, y_ref, d_ref, b_ref, w_ref, bo_ref, o_ref):
        a = a_ref[:N, :] + a_ref[NPAD:NPAD + N, :]
        h = jnp.maximum(
            d_ref[...] * (a + y_ref[...]) + b_ref[...], 0.0
        )
        o_ref[...] = (
            jnp.dot(h, w_ref[...], preferred_element_type=jnp.float32) + bo_ref[...]
        )

    return pl.pallas_call(
        body, out_shape=jax.ShapeDtypeStruct((N, 2), jnp.float32)
    )(agg_parts, y, dis_b, b, Whp, bhp)


def kernel(x, edge_index, W1, b1, W2, b2, Wd, bd, Wp, bp):
    ei = edge_index.astype(jnp.int32)
    src = ei[0]
    dst = ei[1]
    pad = EPAD - E
    fill = jnp.arange(pad, dtype=jnp.int32)
    srcp = jnp.concatenate([src, fill % N]).reshape(NW * C, CH)
    dstp = jnp.concatenate([dst, N + fill % (NPAD - N)]).reshape(NW * C, CH)

    deg_parts = _sc_degree(dstp)
    xw1 = _tc_matmul(x, W1)
    dis_b, y1 = _tc_norm_scale(deg_parts, xw1)

    a1 = _sc_aggregate(y1, srcp, dstp)
    y2 = _tc_layer(a1, y1, dis_b, b1.reshape(1, D), W2)

    a2 = _sc_aggregate(y2, srcp, dstp)
    whp = jnp.concatenate([Wd, Wp], axis=1)
    bhp = jnp.concatenate([bd, bp]).reshape(1, 2)
    out = _tc_final(a2, y2, dis_b, b2.reshape(1, D), whp, bhp)
    return out[:, :1], out[:, 1:2]


# speedup vs baseline: 20.9517x; 1.0416x over previous
import functools

import jax
import jax.numpy as jnp
from jax import lax
from jax.experimental import pallas as pl
from jax.experimental.pallas import tpu as pltpu
from jax.experimental.pallas import tpu_sc as plsc

N = 10000
E = 320000
D = 128
NC = 2
NS = 16
NW = NC * NS
CHD = 128
CD = 80
HP = 2
ZB = 64
EPAD = NW * CD * CHD
NPAD = 10240
RPW = NPAD // NS
ZR = 16


def _mesh():
    return plsc.VectorSubcoreMesh(core_axis_name="c", subcore_axis_name="s")


def _sc_degree(dst_rows):
    @functools.partial(
        pl.kernel,
        out_type=jax.ShapeDtypeStruct((NC * NPAD, D), jnp.float32),
        mesh=_mesh(),
        scratch_types=[
            pltpu.VMEM((CD, CHD), jnp.int32),
            pltpu.VMEM((CHD, D), jnp.float32),
            pltpu.VMEM((ZR, D), jnp.float32),
            pltpu.VMEM_SHARED((NPAD, D), jnp.float32),
            pltpu.SemaphoreType.DMA,
            pltpu.SemaphoreType.DMA,
        ],
    )
    def deg_kernel(dst_hbm, out_hbm, dstv, onesv, zerov, degs, zsem, sem):
        c = lax.axis_index("c")
        s = lax.axis_index("s")
        w = c * NS + s

        pltpu.async_copy(dst_hbm.at[pl.ds(w * CD, CD)], dstv, sem)

        @pl.loop(0, CHD)
        def _(i):
            @pl.loop(0, D, step=16)
            def _(k):
                onesv[i, pl.ds(k, 16)] = jnp.ones((16,), jnp.float32)

        @pl.loop(0, ZR)
        def _(i):
            @pl.loop(0, D, step=16)
            def _(k):
                zerov[i, pl.ds(k, 16)] = jnp.zeros((16,), jnp.float32)

        @pl.loop(0, RPW // ZR)
        def _(i):
            pltpu.async_copy(zerov, degs.at[pl.ds(s * RPW + i * ZR, ZR)], zsem)

        @pl.loop(0, RPW // ZR)
        def _(i):
            pltpu.make_async_copy(
                zerov, degs.at[pl.ds(s * RPW + i * ZR, ZR)], zsem
            ).wait()

        pltpu.make_async_copy(dst_hbm.at[pl.ds(w * CD, CD)], dstv, sem).wait()
        plsc.subcore_barrier()

        @pl.loop(0, CD)
        def _(j):
            pltpu.async_copy(onesv, degs.at[dstv.at[j]], sem, add=True)

        @pl.loop(0, CD)
        def _(j):
            pltpu.make_async_copy(onesv, degs.at[dstv.at[j]], sem).wait()

        plsc.subcore_barrier()
        pltpu.sync_copy(
            degs.at[pl.ds(s * RPW, RPW)],
            out_hbm.at[pl.ds(c * NPAD + s * RPW, RPW)],
        )

    return deg_kernel(dst_rows)


def _sc_aggregate(y, zeros_hbm, src_rows, dst_rows):
    CP = CD // HP

    @functools.partial(
        pl.kernel,
        out_type=jax.ShapeDtypeStruct((NC * NPAD, D), jnp.float32),
        mesh=_mesh(),
        scratch_types=[
            pltpu.VMEM((CP, CHD), jnp.int32),
            pltpu.VMEM((CP, CHD), jnp.int32),
            pltpu.VMEM((CHD, D), jnp.float32),
            pltpu.VMEM((CHD, D), jnp.float32),
            pltpu.VMEM_SHARED((NPAD, D), jnp.float32),
            pltpu.SemaphoreType.DMA,
            pltpu.SemaphoreType.DMA,
            pltpu.SemaphoreType.DMA,
            pltpu.SemaphoreType.DMA,
            pltpu.SemaphoreType.DMA,
        ],
    )
    def agg_kernel(
        y_hbm, z_hbm, src_hbm, dst_hbm, out_hbm,
        srcv, dstv, rows0, rows1, aggs,
        gsem0, gsem1, ssem0, ssem1, zsem,
    ):
        c = lax.axis_index("c")
        s = lax.axis_index("s")
        w = c * NS + s

        @pl.loop(0, RPW // ZB)
        def _(i):
            pltpu.async_copy(z_hbm, aggs.at[pl.ds(s * RPW + i * ZB, ZB)], zsem)

        @pl.loop(0, RPW // ZB)
        def _(i):
            pltpu.make_async_copy(
                z_hbm, aggs.at[pl.ds(s * RPW + i * ZB, ZB)], zsem
            ).wait()

        plsc.subcore_barrier()

        for p in range(HP):
            base = w * CD + p * CP
            pltpu.sync_copy(src_hbm.at[pl.ds(base, CP)], srcv)
            pltpu.sync_copy(dst_hbm.at[pl.ds(base, CP)], dstv)

            pltpu.async_copy(y_hbm.at[srcv.at[0]], rows0, gsem0)
            pltpu.async_copy(y_hbm.at[srcv.at[1]], rows1, gsem1)

            @pl.loop(0, CP, step=2)
            def _(j):
                pltpu.make_async_copy(y_hbm.at[srcv.at[j]], rows0, gsem0).wait()
                pltpu.async_copy(rows0, aggs.at[dstv.at[j]], ssem0, add=True)
                pltpu.make_async_copy(y_hbm.at[srcv.at[j + 1]], rows1, gsem1).wait()
                pltpu.async_copy(rows1, aggs.at[dstv.at[j + 1]], ssem1, add=True)
                pltpu.make_async_copy(rows0, aggs.at[dstv.at[j]], ssem0).wait()

                @pl.when(j + 2 < CP)
                def _():
                    pltpu.async_copy(y_hbm.at[srcv.at[j + 2]], rows0, gsem0)

                pltpu.make_async_copy(rows1, aggs.at[dstv.at[j + 1]], ssem1).wait()

                @pl.when(j + 3 < CP)
                def _():
                    pltpu.async_copy(y_hbm.at[srcv.at[j + 3]], rows1, gsem1)

        plsc.subcore_barrier()
        pltpu.sync_copy(
            aggs.at[pl.ds(s * RPW, RPW)],
            out_hbm.at[pl.ds(c * NPAD + s * RPW, RPW)],
        )

    return agg_kernel(y, zeros_hbm, src_rows, dst_rows)


def _tc_matmul(x, W):
    def body(x_ref, w_ref, o_ref):
        o_ref[...] = jnp.dot(
            x_ref[...], w_ref[...], preferred_element_type=jnp.float32
        )

    return pl.pallas_call(
        body, out_shape=jax.ShapeDtypeStruct((N, D), jnp.float32)
    )(x, W)


def _tc_norm_scale(deg_parts, xw):
    def body(p_ref, xw_ref, dis_ref, y_ref):
        p = p_ref[...]
        deg = 1.0 + p[:N, :1] + p[NPAD:NPAD + N, :1]
        dis_b = jnp.broadcast_to(lax.rsqrt(deg), (N, D))
        dis_ref[...] = dis_b
        y_ref[...] = dis_b * xw_ref[...]

    return pl.pallas_call(
        body,
        out_shape=[
            jax.ShapeDtypeStruct((N, D), jnp.float32),
            jax.ShapeDtypeStruct((N, D), jnp.float32),
        ],
    )(deg_parts, xw)


def _tc_layer(agg_parts, y, dis_b, b, W):
    def body(a_ref, y_ref, d_ref, b_ref, w_ref, o_ref):
        d = d_ref[...]
        a = a_ref[:N, :] + a_ref[NPAD:NPAD + N, :]
        h = jnp.maximum(d * (a + y_ref[...]) + b_ref[...], 0.0)
        o_ref[...] = d * jnp.dot(h, w_ref[...], preferred_element_type=jnp.float32)

    return pl.pallas_call(
        body, out_shape=jax.ShapeDtypeStruct((N, D), jnp.float32)
    )(agg_parts, y, dis_b, b, W)


def _tc_final(agg_parts, y, dis_b, b, Whp, bhp):
    def body(a_ref, y_ref, d_ref, b_ref, w_ref, bo_ref, o_ref):
        a = a_ref[:N, :] + a_ref[NPAD:NPAD + N, :]
        h = jnp.maximum(
            d_ref[...] * (a + y_ref[...]) + b_ref[...], 0.0
        )
        o_ref[...] = (
            jnp.dot(h, w_ref[...], preferred_element_type=jnp.float32) + bo_ref[...]
        )

    return pl.pallas_call(
        body, out_shape=jax.ShapeDtypeStruct((N, 2), jnp.float32)
    )(agg_parts, y, dis_b, b, Whp, bhp)


def kernel(x, edge_index, W1, b1, W2, b2, Wd, bd, Wp, bp):
    ei = edge_index.astype(jnp.int32)
    src = ei[0]
    dst = ei[1]
    pad = EPAD - E
    fill = jnp.arange(pad, dtype=jnp.int32)
    srcp = jnp.concatenate([src, fill % N]).reshape(NW * CD, CHD)
    dstp = jnp.concatenate([dst, N + fill % (NPAD - N)]).reshape(NW * CD, CHD)
    zblk = jnp.zeros((ZB, D), jnp.float32)

    deg_parts = _sc_degree(dstp)
    xw1 = _tc_matmul(x, W1)
    dis_b, y1 = _tc_norm_scale(deg_parts, xw1)

    a1 = _sc_aggregate(y1, zblk, srcp, dstp)
    y2 = _tc_layer(a1, y1, dis_b, b1.reshape(1, D), W2)

    a2 = _sc_aggregate(y2, zblk, srcp, dstp)
    whp = jnp.concatenate([Wd, Wp], axis=1)
    bhp = jnp.concatenate([bd, bp]).reshape(1, 2)
    out = _tc_final(a2, y2, dis_b, b2.reshape(1, D), whp, bhp)
    return out[:, :1], out[:, 1:2]


# speedup vs baseline: 23.7310x; 1.1327x over previous
import functools

import jax
import jax.numpy as jnp
from jax import lax
from jax.experimental import pallas as pl
from jax.experimental.pallas import tpu as pltpu
from jax.experimental.pallas import tpu_sc as plsc

N = 10000
E = 320000
D = 128
NC = 2
NS = 16
NW = NC * NS
CHD = 128
CD = 80
HP = 4
NB = 4
CHA = 64
ZB = 64
EPAD = NW * CD * CHD
NPAD = 10240
RPW = NPAD // NS
ZR = 16


def _mesh():
    return plsc.VectorSubcoreMesh(core_axis_name="c", subcore_axis_name="s")


def _sc_degree(dst_rows):
    @functools.partial(
        pl.kernel,
        out_type=jax.ShapeDtypeStruct((NC * NPAD, D), jnp.float32),
        mesh=_mesh(),
        scratch_types=[
            pltpu.VMEM((CD, CHD), jnp.int32),
            pltpu.VMEM((CHD, D), jnp.float32),
            pltpu.VMEM((ZR, D), jnp.float32),
            pltpu.VMEM_SHARED((NPAD, D), jnp.float32),
            pltpu.SemaphoreType.DMA,
            pltpu.SemaphoreType.DMA,
        ],
    )
    def deg_kernel(dst_hbm, out_hbm, dstv, onesv, zerov, degs, zsem, sem):
        c = lax.axis_index("c")
        s = lax.axis_index("s")
        w = c * NS + s

        pltpu.async_copy(dst_hbm.at[pl.ds(w * CD, CD)], dstv, sem)

        @pl.loop(0, CHD)
        def _(i):
            @pl.loop(0, D, step=16)
            def _(k):
                onesv[i, pl.ds(k, 16)] = jnp.ones((16,), jnp.float32)

        @pl.loop(0, ZR)
        def _(i):
            @pl.loop(0, D, step=16)
            def _(k):
                zerov[i, pl.ds(k, 16)] = jnp.zeros((16,), jnp.float32)

        @pl.loop(0, RPW // ZR)
        def _(i):
            pltpu.async_copy(zerov, degs.at[pl.ds(s * RPW + i * ZR, ZR)], zsem)

        @pl.loop(0, RPW // ZR)
        def _(i):
            pltpu.make_async_copy(
                zerov, degs.at[pl.ds(s * RPW + i * ZR, ZR)], zsem
            ).wait()

        pltpu.make_async_copy(dst_hbm.at[pl.ds(w * CD, CD)], dstv, sem).wait()
        plsc.subcore_barrier()

        @pl.loop(0, CD)
        def _(j):
            pltpu.async_copy(onesv, degs.at[dstv.at[j]], sem, add=True)

        @pl.loop(0, CD)
        def _(j):
            pltpu.make_async_copy(onesv, degs.at[dstv.at[j]], sem).wait()

        plsc.subcore_barrier()
        pltpu.sync_copy(
            degs.at[pl.ds(s * RPW, RPW)],
            out_hbm.at[pl.ds(c * NPAD + s * RPW, RPW)],
        )

    return deg_kernel(dst_rows)


def _sc_aggregate(y, zeros_hbm, src_rows, dst_rows):
    CP = (CD * CHD) // (HP * CHA)

    @functools.partial(
        pl.kernel,
        out_type=jax.ShapeDtypeStruct((NC * NPAD, D), jnp.float32),
        mesh=_mesh(),
        scratch_types=[
            pltpu.VMEM((CP, CHA), jnp.int32),
            pltpu.VMEM((CP, CHA), jnp.int32),
        ]
        + [pltpu.VMEM((CHA, D), jnp.float32) for _ in range(NB)]
        + [
            pltpu.VMEM_SHARED((NPAD, D), jnp.float32),
            pltpu.SemaphoreType.DMA,
        ]
        + [pltpu.SemaphoreType.DMA for _ in range(NB)]
        + [pltpu.SemaphoreType.DMA for _ in range(NB)],
    )
    def agg_kernel(y_hbm, z_hbm, src_hbm, dst_hbm, out_hbm, srcv, dstv, *rest):
        rows = rest[:NB]
        aggs = rest[NB]
        zsem = rest[NB + 1]
        gsems = rest[NB + 2 : NB + 2 + NB]
        ssems = rest[NB + 2 + NB :]
        c = lax.axis_index("c")
        s = lax.axis_index("s")
        w = c * NS + s

        @pl.loop(0, RPW // ZB)
        def _(i):
            pltpu.async_copy(z_hbm, aggs.at[pl.ds(s * RPW + i * ZB, ZB)], zsem)

        @pl.loop(0, RPW // ZB)
        def _(i):
            pltpu.make_async_copy(
                z_hbm, aggs.at[pl.ds(s * RPW + i * ZB, ZB)], zsem
            ).wait()

        plsc.subcore_barrier()

        for p in range(HP):
            base = w * CP * HP + p * CP
            pltpu.sync_copy(src_hbm.at[pl.ds(base, CP)], srcv)
            pltpu.sync_copy(dst_hbm.at[pl.ds(base, CP)], dstv)

            for b in range(NB):
                pltpu.async_copy(y_hbm.at[srcv.at[b]], rows[b], gsems[b])

            @pl.loop(0, CP, step=NB)
            def _(j):
                for b in range(NB):
                    pltpu.make_async_copy(
                        y_hbm.at[srcv.at[j + b]], rows[b], gsems[b]
                    ).wait()
                    pltpu.async_copy(
                        rows[b], aggs.at[dstv.at[j + b]], ssems[b], add=True
                    )
                for b in range(NB):
                    pltpu.make_async_copy(
                        rows[b], aggs.at[dstv.at[j + b]], ssems[b]
                    ).wait()

                    @pl.when(j + NB + b < CP)
                    def _():
                        pltpu.async_copy(
                            y_hbm.at[srcv.at[j + NB + b]], rows[b], gsems[b]
                        )

        plsc.subcore_barrier()
        pltpu.sync_copy(
            aggs.at[pl.ds(s * RPW, RPW)],
            out_hbm.at[pl.ds(c * NPAD + s * RPW, RPW)],
        )

    return agg_kernel(y, zeros_hbm, src_rows, dst_rows)


def _tc_matmul(x, W):
    def body(x_ref, w_ref, o_ref):
        o_ref[...] = jnp.dot(
            x_ref[...], w_ref[...], preferred_element_type=jnp.float32
        )

    return pl.pallas_call(
        body, out_shape=jax.ShapeDtypeStruct((N, D), jnp.float32)
    )(x, W)


def _tc_norm_scale(deg_parts, xw):
    def body(p_ref, xw_ref, dis_ref, y_ref):
        p = p_ref[...]
        deg = 1.0 + p[:N, :1] + p[NPAD:NPAD + N, :1]
        dis_b = jnp.broadcast_to(lax.rsqrt(deg), (N, D))
        dis_ref[...] = dis_b
        y_ref[...] = dis_b * xw_ref[...]

    return pl.pallas_call(
        body,
        out_shape=[
            jax.ShapeDtypeStruct((N, D), jnp.float32),
            jax.ShapeDtypeStruct((N, D), jnp.float32),
        ],
    )(deg_parts, xw)


def _tc_layer(agg_parts, y, dis_b, b, W):
    def body(a_ref, y_ref, d_ref, b_ref, w_ref, o_ref):
        d = d_ref[...]
        a = a_ref[:N, :] + a_ref[NPAD:NPAD + N, :]
        h = jnp.maximum(d * (a + y_ref[...]) + b_ref[...], 0.0)
        o_ref[...] = d * jnp.dot(h, w_ref[...], preferred_element_type=jnp.float32)

    return pl.pallas_call(
        body, out_shape=jax.ShapeDtypeStruct((N, D), jnp.float32)
    )(agg_parts, y, dis_b, b, W)


def _tc_final(agg_parts, y, dis_b, b, Whp, bhp):
    def body(a_ref, y_ref, d_ref, b_ref, w_ref, bo_ref, o_ref):
        a = a_ref[:N, :] + a_ref[NPAD:NPAD + N, :]
        h = jnp.maximum(
            d_ref[...] * (a + y_ref[...]) + b_ref[...], 0.0
        )
        o_ref[...] = (
            jnp.dot(h, w_ref[...], preferred_element_type=jnp.float32) + bo_ref[...]
        )

    return pl.pallas_call(
        body, out_shape=jax.ShapeDtypeStruct((N, 2), jnp.float32)
    )(agg_parts, y, dis_b, b, Whp, bhp)


def kernel(x, edge_index, W1, b1, W2, b2, Wd, bd, Wp, bp):
    ei = edge_index.astype(jnp.int32)
    src = ei[0]
    dst = ei[1]
    pad = EPAD - E
    fill = jnp.arange(pad, dtype=jnp.int32)
    srcp = jnp.concatenate([src, fill % N]).reshape(NW * CD, CHD)
    dstp = jnp.concatenate([dst, N + fill % (NPAD - N)]).reshape(NW * CD, CHD)
    zblk = jnp.zeros((ZB, D), jnp.float32)

    deg_parts = _sc_degree(dstp)
    xw1 = _tc_matmul(x, W1)
    dis_b, y1 = _tc_norm_scale(deg_parts, xw1)

    srcpa = srcp.reshape(-1, CHA)
    dstpa = dstp.reshape(-1, CHA)
    a1 = _sc_aggregate(y1, zblk, srcpa, dstpa)
    y2 = _tc_layer(a1, y1, dis_b, b1.reshape(1, D), W2)

    a2 = _sc_aggregate(y2, zblk, srcpa, dstpa)
    whp = jnp.concatenate([Wd, Wp], axis=1)
    bhp = jnp.concatenate([bd, bp]).reshape(1, 2)
    out = _tc_final(a2, y2, dis_b, b2.reshape(1, D), whp, bhp)
    return out[:, :1], out[:, 1:2]
